# Initial kernel scaffold; baseline (speedup 1.0000x reference)
import jax
import jax.numpy as jnp
from jax.experimental import pallas as pl


def kernel(x, edge_index, W1s, W1d, a1s, a1d, b1, W2s, W2d, a2s, a2d, b2, W3s, W3d, a3s, a3d, b3):
    raise NotImplementedError("write your pallas kernel here")


# speedup vs baseline: 11.6887x; 11.6887x over previous
import functools

import jax
import jax.numpy as jnp
from jax import lax
from jax.experimental import pallas as pl
from jax.experimental.pallas import tpu as pltpu
from jax.experimental.pallas import tpu_sc as plsc

NN = 10000
EE = 320000
DD = 128

L = 16
NCORES = 2
NSUB = 16
NW = NCORES * NSUB
RN = 320
NPAD = NW * RN
CH_A = 512
CH_B = 128
EPAD = EE + CH_A

_NEG = -1e30


def _iota16():
    return lax.iota(jnp.int32, L)


def _vgather(v, idx):
    dnums = lax.GatherDimensionNumbers(
        offset_dims=(), collapsed_slice_dims=(0,), start_index_map=(0,))
    return lax.gather(v, idx[:, None], dnums, (1,),
                      mode=lax.GatherScatterMode.PROMISE_IN_BOUNDS)


def _shift_up(v, k, fill):
    it = _iota16()
    src = jnp.maximum(it - k, 0)
    sh = _vgather(v, src)
    return jnp.where(it >= k, sh, fill)


_TC_BLK = 1024


def _tc_body(x_ref, ws_ref, wd_ref, avs_ref, avd_ref, xs_ref, as_ref, ad_ref):
    x = x_ref[...]
    xs = jnp.dot(x, ws_ref[...], preferred_element_type=jnp.float32)
    xd = jnp.dot(x, wd_ref[...], preferred_element_type=jnp.float32)
    xs_ref[...] = xs
    as_ref[...] = jnp.dot(xs, avs_ref[...], preferred_element_type=jnp.float32)
    ad_ref[...] = jnp.dot(xd, avd_ref[...], preferred_element_type=jnp.float32)


def _tc_stage(x, Ws, Wd, a_s, a_d):
    grid = (NPAD // _TC_BLK,)
    out = pl.pallas_call(
        _tc_body,
        grid=grid,
        in_specs=[
            pl.BlockSpec((_TC_BLK, DD), lambda i: (i, 0)),
            pl.BlockSpec((DD, DD), lambda i: (0, 0)),
            pl.BlockSpec((DD, DD), lambda i: (0, 0)),
            pl.BlockSpec((DD, 1), lambda i: (0, 0)),
            pl.BlockSpec((DD, 1), lambda i: (0, 0)),
        ],
        out_specs=[
            pl.BlockSpec((_TC_BLK, DD), lambda i: (i, 0)),
            pl.BlockSpec((_TC_BLK, 1), lambda i: (i, 0)),
            pl.BlockSpec((_TC_BLK, 1), lambda i: (i, 0)),
        ],
        out_shape=[
            jax.ShapeDtypeStruct((NPAD, DD), jnp.float32),
            jax.ShapeDtypeStruct((NPAD, 1), jnp.float32),
            jax.ShapeDtypeStruct((NPAD, 1), jnp.float32),
        ],
    )(x, Ws, Wd, a_s[:, None], a_d[:, None])
    xs, asrc, adst = out
    return xs, asrc[:, 0], adst[:, 0]


def _seg_merge(l, dst_v):
    pm = l
    ps = jnp.ones((L,), jnp.float32)
    it = _iota16()
    for k in (1, 2, 4, 8):
        m_sh = _shift_up(pm, k, _NEG)
        s_sh = _shift_up(ps, k, 0.0)
        d_sh = _shift_up(dst_v, k, -1)
        same = (d_sh == dst_v) & (it >= k)
        mn = jnp.maximum(pm, jnp.where(same, m_sh, _NEG))
        sn = ps * jnp.exp(pm - mn) + jnp.where(same, s_sh * jnp.exp(m_sh - mn), 0.0)
        pm, ps = mn, sn
    return pm, ps


def _run_end_mask(dst_v):
    it = _iota16()
    nxt = jnp.minimum(it + 1, L - 1)
    dn = _vgather(dst_v, nxt)
    return (dst_v != dn) | (it == L - 1)


def _sc_body(relu, xs_hbm, asrc_hbm, adst_hbm, src_hbm, dst_hbm, starts_hbm,
             b_hbm, out_hbm, asrc_v, adst_v, m_v, s_v, out_v, srcA, dstA,
             idxB, dstB, alphaB, dlocB, rows_v, starts_v, bias_v, sem):
    cid = lax.axis_index("c")
    sid = lax.axis_index("s")
    wid = sid * NCORES + cid
    node_lo = wid * RN

    pltpu.sync_copy(asrc_hbm, asrc_v)
    pltpu.sync_copy(adst_hbm.at[pl.ds(node_lo, RN)], adst_v)
    pltpu.sync_copy(starts_hbm, starts_v)
    pltpu.sync_copy(b_hbm, bias_v)

    e_lo = starts_v[pl.ds(wid, L)][0]
    e_hi = starts_v[pl.ds(wid + 1, L)][0]
    base = (e_lo // L) * L

    it = _iota16()
    negv = jnp.full((L,), _NEG, jnp.float32)
    zerov = jnp.zeros((L,), jnp.float32)

    def init_ms(i, c):
        m_v[pl.ds(i * L, L)] = negv
        s_v[pl.ds(i * L, L)] = zerov
        return c

    lax.fori_loop(0, RN // L, init_ms, 0)

    def init_out(i, c):
        for ch in range(DD // L):
            out_v[i, pl.ds(ch * L, L)] = bias_v[pl.ds(ch * L, L)]
        return c

    lax.fori_loop(0, RN, init_out, 0)

    n_a = (e_hi - base + CH_A - 1) // CH_A

    def passA(t, c):
        off = base + t * CH_A
        pltpu.sync_copy(src_hbm.at[pl.ds(off, CH_A)], srcA)
        pltpu.sync_copy(dst_hbm.at[pl.ds(off, CH_A)], dstA)

        def vloop(v, cc):
            src_v = srcA[pl.ds(v * L, L)]
            dst_v = dstA[pl.ds(v * L, L)]
            gidx = (off + v * L) + it
            valid = (gidx >= e_lo) & (gidx < e_hi)
            a1 = plsc.load_gather(asrc_v, [jnp.clip(src_v, 0, NPAD - 1)],
                                  mask=valid)
            dlc = jnp.clip(dst_v - node_lo, 0, RN - 1)
            a2 = plsc.load_gather(adst_v, [dlc], mask=valid)
            l = a1 + a2
            l = jnp.where(l > 0, l, 0.2 * l)
            l = jnp.where(valid, l, _NEG)
            pm, ps = _seg_merge(l, dst_v)
            wmask = _run_end_mask(dst_v) & valid
            old_m = plsc.load_gather(m_v, [dlc], mask=wmask)
            old_s = plsc.load_gather(s_v, [dlc], mask=wmask)
            mn = jnp.maximum(old_m, pm)
            sn = old_s * jnp.exp(old_m - mn) + ps * jnp.exp(pm - mn)
            plsc.store_scatter(m_v, [dlc], mn, mask=wmask)
            plsc.store_scatter(s_v, [dlc], sn, mask=wmask)
            return cc

        lax.fori_loop(0, CH_A // L, vloop, 0)
        return c

    lax.fori_loop(0, n_a, passA, 0)

    n_b = (e_hi - base + CH_B - 1) // CH_B

    def passB(t, c):
        off = base + t * CH_B
        pltpu.sync_copy(src_hbm.at[pl.ds(off, CH_B)], idxB)
        pltpu.sync_copy(dst_hbm.at[pl.ds(off, CH_B)], dstB)
        cp = pltpu.make_async_copy(xs_hbm.at[idxB], rows_v, sem)
        cp.start()
        for v in range(CH_B // L):
            src_v = idxB[pl.ds(v * L, L)]
            dst_v = dstB[pl.ds(v * L, L)]
            gidx = (off + v * L) + it
            valid = (gidx >= e_lo) & (gidx < e_hi)
            a1 = plsc.load_gather(asrc_v, [jnp.clip(src_v, 0, NPAD - 1)],
                                  mask=valid)
            dlc = jnp.clip(dst_v - node_lo, 0, RN - 1)
            a2 = plsc.load_gather(adst_v, [dlc], mask=valid)
            l = a1 + a2
            l = jnp.where(l > 0, l, 0.2 * l)
            m_g = plsc.load_gather(m_v, [dlc], mask=valid)
            s_g = plsc.load_gather(s_v, [dlc], mask=valid)
            alpha = jnp.exp(l - m_g) / (s_g + 1e-16)
            alpha = jnp.where(valid, alpha, 0.0)
            alphaB[pl.ds(v * L, L)] = alpha
            dlocB[pl.ds(v * L, L)] = dlc
        cp.wait()

        def eloop(e, cc):
            a = alphaB[pl.ds(e, L)][0]
            d = dlocB[pl.ds(e, L)][0]
            av = lax.broadcast(a, (L,))
            for ch in range(DD // L):
                r = rows_v[e, pl.ds(ch * L, L)]
                plsc.addupdate(out_v.at[d, pl.ds(ch * L, L)], av * r)
            return cc

        lax.fori_loop(0, CH_B, eloop, 0)
        return c

    lax.fori_loop(0, n_b, passB, 0)

    if relu:
        def fin(i, c):
            for ch in range(DD // L):
                v = out_v[i, pl.ds(ch * L, L)]
                out_v[i, pl.ds(ch * L, L)] = jnp.maximum(v, 0.0)
            return c

        lax.fori_loop(0, RN, fin, 0)

    pltpu.sync_copy(out_v, out_hbm.at[pl.ds(node_lo, RN)])


def _make_sc(relu):
    mesh = plsc.VectorSubcoreMesh(core_axis_name="c", subcore_axis_name="s",
                                  num_cores=NCORES, num_subcores=NSUB)
    return pl.kernel(
        functools.partial(_sc_body, relu),
        out_type=jax.ShapeDtypeStruct((NPAD, DD), jnp.float32),
        mesh=mesh,
        compiler_params=pltpu.CompilerParams(needs_layout_passes=False),
        scratch_types=[
            pltpu.VMEM((NPAD,), jnp.float32),
            pltpu.VMEM((RN,), jnp.float32),
            pltpu.VMEM((RN,), jnp.float32),
            pltpu.VMEM((RN,), jnp.float32),
            pltpu.VMEM((RN, DD), jnp.float32),
            pltpu.VMEM((CH_A,), jnp.int32),
            pltpu.VMEM((CH_A,), jnp.int32),
            pltpu.VMEM((CH_B,), jnp.int32),
            pltpu.VMEM((CH_B,), jnp.int32),
            pltpu.VMEM((CH_B + L,), jnp.float32),
            pltpu.VMEM((CH_B + L,), jnp.int32),
            pltpu.VMEM((CH_B, DD), jnp.float32),
            pltpu.VMEM((48,), jnp.int32),
            pltpu.VMEM((DD,), jnp.float32),
            pltpu.SemaphoreType.DMA,
        ],
    )


_sc_relu = _make_sc(True)
_sc_last = _make_sc(False)


def kernel(x, edge_index, W1s, W1d, a1s, a1d, b1, W2s, W2d, a2s, a2d, b2,
           W3s, W3d, a3s, a3d, b3):
    src = edge_index[0].astype(jnp.int32)
    dst = edge_index[1].astype(jnp.int32)
    order = jnp.argsort(dst)
    src_s = jnp.take(src, order)
    dst_s = jnp.take(dst, order)
    src_s = jnp.concatenate([src_s, jnp.zeros((EPAD - EE,), jnp.int32)])
    dst_s = jnp.concatenate([dst_s, jnp.zeros((EPAD - EE,), jnp.int32)])
    bounds = jnp.arange(0, NPAD + RN, RN, dtype=jnp.int32)
    starts = jnp.searchsorted(dst_s[:EE], bounds).astype(jnp.int32)
    starts = jnp.concatenate(
        [starts, jnp.full((48 - starts.shape[0],), EE, jnp.int32)])

    h = jnp.zeros((NPAD, DD), jnp.float32).at[:NN].set(x)
    layers = [
        (W1s, W1d, a1s, a1d, b1, _sc_relu),
        (W2s, W2d, a2s, a2d, b2, _sc_relu),
        (W3s, W3d, a3s, a3d, b3, _sc_last),
    ]
    for Ws, Wd, a_s, a_d, b, sc in layers:
        xs, asrc, adst = _tc_stage(h, Ws, Wd, a_s, a_d)
        h = sc(xs, asrc, adst, src_s, dst_s, starts, b)
    return h[:NN]


# speedup vs baseline: 14.7178x; 1.2591x over previous
import functools

import jax
import jax.numpy as jnp
from jax import lax
from jax.experimental import pallas as pl
from jax.experimental.pallas import tpu as pltpu
from jax.experimental.pallas import tpu_sc as plsc

NN = 10000
EE = 320000
DD = 128

L = 16
NCORES = 2
NSUB = 16
NW = NCORES * NSUB
RN = 320
NPAD = NW * RN
CH_A = 512
CH_B = 128
EPAD = EE + CH_A

_NEG = -1e30


def _iota16():
    return lax.iota(jnp.int32, L)


def _vgather(v, idx):
    dnums = lax.GatherDimensionNumbers(
        offset_dims=(), collapsed_slice_dims=(0,), start_index_map=(0,))
    return lax.gather(v, idx[:, None], dnums, (1,),
                      mode=lax.GatherScatterMode.PROMISE_IN_BOUNDS)


def _shift_up(v, k, fill):
    it = _iota16()
    src = jnp.maximum(it - k, 0)
    sh = _vgather(v, src)
    return jnp.where(it >= k, sh, fill)


_TC_BLK = 1024


def _tc_body(x_ref, ws_ref, wd_ref, avs_ref, avd_ref, xs_ref, as_ref, ad_ref):
    x = x_ref[...]
    xs = jnp.dot(x, ws_ref[...], preferred_element_type=jnp.float32)
    xd = jnp.dot(x, wd_ref[...], preferred_element_type=jnp.float32)
    xs_ref[...] = xs
    as_ref[...] = jnp.dot(xs, avs_ref[...], preferred_element_type=jnp.float32)
    ad_ref[...] = jnp.dot(xd, avd_ref[...], preferred_element_type=jnp.float32)


def _tc_stage(x, Ws, Wd, a_s, a_d):
    grid = (NPAD // _TC_BLK,)
    out = pl.pallas_call(
        _tc_body,
        grid=grid,
        in_specs=[
            pl.BlockSpec((_TC_BLK, DD), lambda i: (i, 0)),
            pl.BlockSpec((DD, DD), lambda i: (0, 0)),
            pl.BlockSpec((DD, DD), lambda i: (0, 0)),
            pl.BlockSpec((DD, 1), lambda i: (0, 0)),
            pl.BlockSpec((DD, 1), lambda i: (0, 0)),
        ],
        out_specs=[
            pl.BlockSpec((_TC_BLK, DD), lambda i: (i, 0)),
            pl.BlockSpec((_TC_BLK, 1), lambda i: (i, 0)),
            pl.BlockSpec((_TC_BLK, 1), lambda i: (i, 0)),
        ],
        out_shape=[
            jax.ShapeDtypeStruct((NPAD, DD), jnp.float32),
            jax.ShapeDtypeStruct((NPAD, 1), jnp.float32),
            jax.ShapeDtypeStruct((NPAD, 1), jnp.float32),
        ],
    )(x, Ws, Wd, a_s[:, None], a_d[:, None])
    xs, asrc, adst = out
    return xs, asrc[:, 0], adst[:, 0]


def _seg_merge(l, dst_v):
    pm = l
    ps = jnp.ones((L,), jnp.float32)
    it = _iota16()
    for k in (1, 2, 4, 8):
        m_sh = _shift_up(pm, k, _NEG)
        s_sh = _shift_up(ps, k, 0.0)
        d_sh = _shift_up(dst_v, k, -1)
        same = (d_sh == dst_v) & (it >= k)
        mn = jnp.maximum(pm, jnp.where(same, m_sh, _NEG))
        sn = ps * jnp.exp(pm - mn) + jnp.where(same, s_sh * jnp.exp(m_sh - mn), 0.0)
        pm, ps = mn, sn
    return pm, ps


def _run_end_mask(dst_v):
    it = _iota16()
    nxt = jnp.minimum(it + 1, L - 1)
    dn = _vgather(dst_v, nxt)
    return (dst_v != dn) | (it == L - 1)


def _sc_body(relu, xs_hbm, asrc_hbm, adst_hbm, src_hbm, dst_hbm, starts_hbm,
             b_hbm, out_hbm, asrc_v, adst_v, m_v, s_v, out_v, srcA, dstA,
             idxB0, idxB1, dstB0, dstB1, rows0, rows1, starts_v, bias_v,
             sem0, sem1):
    cid = lax.axis_index("c")
    sid = lax.axis_index("s")
    wid = sid * NCORES + cid
    node_lo = wid * RN

    pltpu.sync_copy(asrc_hbm, asrc_v)
    pltpu.sync_copy(adst_hbm.at[pl.ds(node_lo, RN)], adst_v)
    pltpu.sync_copy(starts_hbm, starts_v)
    pltpu.sync_copy(b_hbm, bias_v)

    e_lo = starts_v[pl.ds(wid, L)][0]
    e_hi = starts_v[pl.ds(wid + 1, L)][0]
    base = (e_lo // L) * L

    it = _iota16()
    negv = jnp.full((L,), _NEG, jnp.float32)
    zerov = jnp.zeros((L,), jnp.float32)

    def init_ms(i, c):
        m_v[pl.ds(i * L, L)] = negv
        s_v[pl.ds(i * L, L)] = zerov
        return c

    lax.fori_loop(0, RN // L, init_ms, 0)

    def init_out(i, c):
        for ch in range(DD // L):
            out_v[i, pl.ds(ch * L, L)] = bias_v[pl.ds(ch * L, L)]
        return c

    lax.fori_loop(0, RN, init_out, 0)

    n_a = (e_hi - base + CH_A - 1) // CH_A

    def passA(t, c):
        off = base + t * CH_A
        pltpu.sync_copy(src_hbm.at[pl.ds(off, CH_A)], srcA)
        pltpu.sync_copy(dst_hbm.at[pl.ds(off, CH_A)], dstA)

        def vloop(v, cc):
            src_v = srcA[pl.ds(v * L, L)]
            dst_v = dstA[pl.ds(v * L, L)]
            gidx = (off + v * L) + it
            valid = (gidx >= e_lo) & (gidx < e_hi)
            a1 = plsc.load_gather(asrc_v, [jnp.clip(src_v, 0, NPAD - 1)],
                                  mask=valid)
            dlc = jnp.clip(dst_v - node_lo, 0, RN - 1)
            a2 = plsc.load_gather(adst_v, [dlc], mask=valid)
            l = a1 + a2
            l = jnp.where(l > 0, l, 0.2 * l)
            l = jnp.where(valid, l, _NEG)
            pm, ps = _seg_merge(l, dst_v)
            wmask = _run_end_mask(dst_v) & valid
            old_m = plsc.load_gather(m_v, [dlc], mask=wmask)
            old_s = plsc.load_gather(s_v, [dlc], mask=wmask)
            mn = jnp.maximum(old_m, pm)
            sn = old_s * jnp.exp(old_m - mn) + ps * jnp.exp(pm - mn)
            plsc.store_scatter(m_v, [dlc], mn, mask=wmask)
            plsc.store_scatter(s_v, [dlc], sn, mask=wmask)
            return cc

        lax.fori_loop(0, CH_A // L, vloop, 0)
        return c

    lax.fori_loop(0, n_a, passA, 0)

    n_b = (e_hi - base + CH_B - 1) // CH_B
    idxBs = (idxB0, idxB1)
    dstBs = (dstB0, dstB1)
    rowss = (rows0, rows1)
    sems = (sem0, sem1)

    def _fetchB(t, b):
        off = base + t * CH_B
        pltpu.sync_copy(src_hbm.at[pl.ds(off, CH_B)], idxBs[b])
        pltpu.sync_copy(dst_hbm.at[pl.ds(off, CH_B)], dstBs[b])
        pltpu.make_async_copy(xs_hbm.at[idxBs[b]], rowss[b], sems[b]).start()

    @pl.when(n_b > 0)
    def _():
        _fetchB(0, 0)

    @pl.when(n_b > 1)
    def _():
        _fetchB(1, 1)

    def _consumeB(t, b):
        pltpu.make_async_copy(xs_hbm.at[idxBs[b]], rowss[b], sems[b]).wait()
        rows = rowss[b]

        def vloop(v, cc):
            src_v = idxBs[b][pl.ds(v * L, L)]
            dst_v = dstBs[b][pl.ds(v * L, L)]
            gidx = (base + t * CH_B + v * L) + it
            valid = (gidx >= e_lo) & (gidx < e_hi)
            a1 = plsc.load_gather(asrc_v, [jnp.clip(src_v, 0, NPAD - 1)],
                                  mask=valid)
            dlc = jnp.clip(dst_v - node_lo, 0, RN - 1)
            a2 = plsc.load_gather(adst_v, [dlc], mask=valid)
            l = a1 + a2
            l = jnp.where(l > 0, l, 0.2 * l)
            m_g = plsc.load_gather(m_v, [dlc], mask=valid)
            s_g = plsc.load_gather(s_v, [dlc], mask=valid)
            alpha = jnp.exp(l - m_g) / (s_g + 1e-16)
            alpha = jnp.where(valid, alpha, 0.0)
            rbase = v * L
            for j in range(L):
                av = lax.broadcast(alpha[j], (L,))
                d = dlc[j]
                for ch in range(DD // L):
                    r = rows[rbase + j, pl.ds(ch * L, L)]
                    plsc.addupdate(out_v.at[d, pl.ds(ch * L, L)], av * r)
            return cc

        lax.fori_loop(0, CH_B // L, vloop, 0)

        @pl.when(t + 2 < n_b)
        def _():
            _fetchB(t + 2, b)

    def passB_pair(p, c):
        for b in range(2):
            t = p * 2 + b

            @pl.when(t < n_b)
            def _():
                _consumeB(t, b)

        return c

    lax.fori_loop(0, (n_b + 1) // 2, passB_pair, 0)

    if relu:
        def fin(i, c):
            for ch in range(DD // L):
                v = out_v[i, pl.ds(ch * L, L)]
                out_v[i, pl.ds(ch * L, L)] = jnp.maximum(v, 0.0)
            return c

        lax.fori_loop(0, RN, fin, 0)

    pltpu.sync_copy(out_v, out_hbm.at[pl.ds(node_lo, RN)])


def _make_sc(relu):
    mesh = plsc.VectorSubcoreMesh(core_axis_name="c", subcore_axis_name="s",
                                  num_cores=NCORES, num_subcores=NSUB)
    return pl.kernel(
        functools.partial(_sc_body, relu),
        out_type=jax.ShapeDtypeStruct((NPAD, DD), jnp.float32),
        mesh=mesh,
        compiler_params=pltpu.CompilerParams(needs_layout_passes=False),
        scratch_types=[
            pltpu.VMEM((NPAD,), jnp.float32),
            pltpu.VMEM((RN,), jnp.float32),
            pltpu.VMEM((RN,), jnp.float32),
            pltpu.VMEM((RN,), jnp.float32),
            pltpu.VMEM((RN, DD), jnp.float32),
            pltpu.VMEM((CH_A,), jnp.int32),
            pltpu.VMEM((CH_A,), jnp.int32),
            pltpu.VMEM((CH_B,), jnp.int32),
            pltpu.VMEM((CH_B,), jnp.int32),
            pltpu.VMEM((CH_B,), jnp.int32),
            pltpu.VMEM((CH_B,), jnp.int32),
            pltpu.VMEM((CH_B, DD), jnp.float32),
            pltpu.VMEM((CH_B, DD), jnp.float32),
            pltpu.VMEM((48,), jnp.int32),
            pltpu.VMEM((DD,), jnp.float32),
            pltpu.SemaphoreType.DMA,
            pltpu.SemaphoreType.DMA,
        ],
    )


_sc_relu = _make_sc(True)
_sc_last = _make_sc(False)


def kernel(x, edge_index, W1s, W1d, a1s, a1d, b1, W2s, W2d, a2s, a2d, b2,
           W3s, W3d, a3s, a3d, b3):
    src = edge_index[0].astype(jnp.int32)
    dst = edge_index[1].astype(jnp.int32)
    order = jnp.argsort(dst)
    src_s = jnp.take(src, order)
    dst_s = jnp.take(dst, order)
    src_s = jnp.concatenate([src_s, jnp.zeros((EPAD - EE,), jnp.int32)])
    dst_s = jnp.concatenate([dst_s, jnp.zeros((EPAD - EE,), jnp.int32)])
    bounds = jnp.arange(0, NPAD + RN, RN, dtype=jnp.int32)
    starts = jnp.searchsorted(dst_s[:EE], bounds).astype(jnp.int32)
    starts = jnp.concatenate(
        [starts, jnp.full((48 - starts.shape[0],), EE, jnp.int32)])

    h = jnp.zeros((NPAD, DD), jnp.float32).at[:NN].set(x)
    layers = [
        (W1s, W1d, a1s, a1d, b1, _sc_relu),
        (W2s, W2d, a2s, a2d, b2, _sc_relu),
        (W3s, W3d, a3s, a3d, b3, _sc_last),
    ]
    for Ws, Wd, a_s, a_d, b, sc in layers:
        xs, asrc, adst = _tc_stage(h, Ws, Wd, a_s, a_d)
        h = sc(xs, asrc, adst, src_s, dst_s, starts, b)
    return h[:NN]


# speedup vs baseline: 23.8521x; 1.6206x over previous
import functools

import jax
import jax.numpy as jnp
from jax import lax
from jax.experimental import pallas as pl
from jax.experimental.pallas import tpu as pltpu
from jax.experimental.pallas import tpu_sc as plsc

NN = 10000
EE = 320000
DD = 128

L = 16
NCORES = 2
NSUB = 16
NW = NCORES * NSUB
RN = 320
NPAD = NW * RN
CH_A = 512
CH_B = 128
EPAD = EE + CH_A

_NEG = -1e30


def _iota16():
    return lax.iota(jnp.int32, L)


def _vgather(v, idx):
    dnums = lax.GatherDimensionNumbers(
        offset_dims=(), collapsed_slice_dims=(0,), start_index_map=(0,))
    return lax.gather(v, idx[:, None], dnums, (1,),
                      mode=lax.GatherScatterMode.PROMISE_IN_BOUNDS)


def _shift_up(v, k, fill):
    it = _iota16()
    src = jnp.maximum(it - k, 0)
    sh = _vgather(v, src)
    return jnp.where(it >= k, sh, fill)


_TC_BLK = 1024


def _tc_body(x_ref, ws_ref, wd_ref, avs_ref, avd_ref, xs_ref, as_ref, ad_ref):
    x = x_ref[...]
    xs = jnp.dot(x, ws_ref[...], preferred_element_type=jnp.float32)
    xd = jnp.dot(x, wd_ref[...], preferred_element_type=jnp.float32)
    xs_ref[...] = xs
    as_ref[...] = jnp.dot(xs, avs_ref[...], preferred_element_type=jnp.float32)
    ad_ref[...] = jnp.dot(xd, avd_ref[...], preferred_element_type=jnp.float32)


def _tc_stage(x, Ws, Wd, a_s, a_d):
    grid = (NPAD // _TC_BLK,)
    out = pl.pallas_call(
        _tc_body,
        grid=grid,
        in_specs=[
            pl.BlockSpec((_TC_BLK, DD), lambda i: (i, 0)),
            pl.BlockSpec((DD, DD), lambda i: (0, 0)),
            pl.BlockSpec((DD, DD), lambda i: (0, 0)),
            pl.BlockSpec((DD, 1), lambda i: (0, 0)),
            pl.BlockSpec((DD, 1), lambda i: (0, 0)),
        ],
        out_specs=[
            pl.BlockSpec((_TC_BLK, DD), lambda i: (i, 0)),
            pl.BlockSpec((_TC_BLK, 1), lambda i: (i, 0)),
            pl.BlockSpec((_TC_BLK, 1), lambda i: (i, 0)),
        ],
        out_shape=[
            jax.ShapeDtypeStruct((NPAD, DD), jnp.float32),
            jax.ShapeDtypeStruct((NPAD, 1), jnp.float32),
            jax.ShapeDtypeStruct((NPAD, 1), jnp.float32),
        ],
    )(x, Ws, Wd, a_s[:, None], a_d[:, None])
    xs, asrc, adst = out
    return xs, asrc[:, 0], adst[:, 0]


def _seg_merge(l, dst_v):
    pm = l
    ps = jnp.ones((L,), jnp.float32)
    it = _iota16()
    for k in (1, 2, 4, 8):
        m_sh = _shift_up(pm, k, _NEG)
        s_sh = _shift_up(ps, k, 0.0)
        d_sh = _shift_up(dst_v, k, -1)
        same = (d_sh == dst_v) & (it >= k)
        mn = jnp.maximum(pm, jnp.where(same, m_sh, _NEG))
        sn = ps * jnp.exp(pm - mn) + jnp.where(same, s_sh * jnp.exp(m_sh - mn), 0.0)
        pm, ps = mn, sn
    return pm, ps


def _run_end_mask(dst_v):
    it = _iota16()
    nxt = jnp.minimum(it + 1, L - 1)
    dn = _vgather(dst_v, nxt)
    return (dst_v != dn) | (it == L - 1)


def _sc_body(relu, xs_hbm, asrc_hbm, adst_hbm, src_hbm, dst_hbm, starts_hbm,
             b_hbm, out_hbm, asrc_v, adst_v, m_v, s_v, out_v, srcA, dstA,
             idxB0, idxB1, dstB0, dstB1, rows0, rows1, starts_v, bias_v,
             sem0, sem1):
    cid = lax.axis_index("c")
    sid = lax.axis_index("s")
    wid = sid * NCORES + cid
    node_lo = wid * RN

    pltpu.sync_copy(asrc_hbm, asrc_v)
    pltpu.sync_copy(adst_hbm.at[pl.ds(node_lo, RN)], adst_v)
    pltpu.sync_copy(starts_hbm, starts_v)
    pltpu.sync_copy(b_hbm, bias_v)

    e_lo = starts_v[pl.ds(wid, L)][0]
    e_hi = starts_v[pl.ds(wid + 1, L)][0]
    base = (e_lo // L) * L

    it = _iota16()
    negv = jnp.full((L,), _NEG, jnp.float32)
    zerov = jnp.zeros((L,), jnp.float32)

    def init_ms(i, c):
        m_v[pl.ds(i * L, L)] = negv
        s_v[pl.ds(i * L, L)] = zerov
        return c

    lax.fori_loop(0, RN // L, init_ms, 0)

    def init_out(i, c):
        for ch in range(DD // L):
            out_v[i, pl.ds(ch * L, L)] = bias_v[pl.ds(ch * L, L)]
        return c

    lax.fori_loop(0, RN, init_out, 0)

    n_a = (e_hi - base + CH_A - 1) // CH_A

    def passA(t, c):
        off = base + t * CH_A
        pltpu.sync_copy(src_hbm.at[pl.ds(off, CH_A)], srcA)
        pltpu.sync_copy(dst_hbm.at[pl.ds(off, CH_A)], dstA)

        def vloop(v, cc):
            src_v = srcA[pl.ds(v * L, L)]
            dst_v = dstA[pl.ds(v * L, L)]
            gidx = (off + v * L) + it
            valid = (gidx >= e_lo) & (gidx < e_hi)
            a1 = plsc.load_gather(asrc_v, [jnp.clip(src_v, 0, NPAD - 1)],
                                  mask=valid)
            dlc = jnp.clip(dst_v - node_lo, 0, RN - 1)
            a2 = plsc.load_gather(adst_v, [dlc], mask=valid)
            l = a1 + a2
            l = jnp.where(l > 0, l, 0.2 * l)
            l = jnp.where(valid, l, _NEG)
            pm, ps = _seg_merge(l, dst_v)
            wmask = _run_end_mask(dst_v) & valid
            old_m = plsc.load_gather(m_v, [dlc], mask=wmask)
            old_s = plsc.load_gather(s_v, [dlc], mask=wmask)
            mn = jnp.maximum(old_m, pm)
            sn = old_s * jnp.exp(old_m - mn) + ps * jnp.exp(pm - mn)
            plsc.store_scatter(m_v, [dlc], mn, mask=wmask)
            plsc.store_scatter(s_v, [dlc], sn, mask=wmask)
            return cc

        lax.fori_loop(0, CH_A // L, vloop, 0)
        return c

    lax.fori_loop(0, n_a, passA, 0)

    n_b = (e_hi - base + CH_B - 1) // CH_B
    idxBs = (idxB0, idxB1)
    dstBs = (dstB0, dstB1)
    rowss = (rows0, rows1)
    sems = (sem0, sem1)

    def _fetchB(t, b):
        off = base + t * CH_B
        pltpu.sync_copy(src_hbm.at[pl.ds(off, CH_B)], idxBs[b])
        pltpu.sync_copy(dst_hbm.at[pl.ds(off, CH_B)], dstBs[b])
        pltpu.make_async_copy(xs_hbm.at[idxBs[b]], rowss[b], sems[b]).start()

    @pl.when(n_b > 0)
    def _():
        _fetchB(0, 0)

    @pl.when(n_b > 1)
    def _():
        _fetchB(1, 1)

    def _consumeB(t, b):
        pltpu.make_async_copy(xs_hbm.at[idxBs[b]], rowss[b], sems[b]).wait()
        rows = rowss[b]

        def vloop(v, cc):
            src_v = idxBs[b][pl.ds(v * L, L)]
            dst_v = dstBs[b][pl.ds(v * L, L)]
            gidx = (base + t * CH_B + v * L) + it
            valid = (gidx >= e_lo) & (gidx < e_hi)
            a1 = plsc.load_gather(asrc_v, [jnp.clip(src_v, 0, NPAD - 1)],
                                  mask=valid)
            dlc = jnp.clip(dst_v - node_lo, 0, RN - 1)
            a2 = plsc.load_gather(adst_v, [dlc], mask=valid)
            l = a1 + a2
            l = jnp.where(l > 0, l, 0.2 * l)
            m_g = plsc.load_gather(m_v, [dlc], mask=valid)
            s_g = plsc.load_gather(s_v, [dlc], mask=valid)
            alpha = jnp.exp(l - m_g) / (s_g + 1e-16)
            alpha = jnp.where(valid, alpha, 0.0)
            rbase = v * L
            endi = _run_end_mask(dst_v).astype(jnp.int32)
            accs = [jnp.zeros((L,), jnp.float32) for _ in range(DD // L)]
            for j in range(L):
                av = lax.broadcast(alpha[j], (L,))
                accs = [acc + av * rows[rbase + j, pl.ds(ch * L, L)]
                        for ch, acc in enumerate(accs)]
                ej = endi[j] != 0
                d = dlc[j]

                @pl.when(ej)
                def _(accs=accs, d=d):
                    for ch in range(DD // L):
                        plsc.addupdate(out_v.at[d, pl.ds(ch * L, L)],
                                       accs[ch])

                eb = lax.broadcast(ej, (L,))
                accs = [jnp.where(eb, 0.0, acc) for acc in accs]
            return cc

        lax.fori_loop(0, CH_B // L, vloop, 0)

        @pl.when(t + 2 < n_b)
        def _():
            _fetchB(t + 2, b)

    def passB_pair(p, c):
        for b in range(2):
            t = p * 2 + b

            @pl.when(t < n_b)
            def _():
                _consumeB(t, b)

        return c

    lax.fori_loop(0, (n_b + 1) // 2, passB_pair, 0)

    if relu:
        def fin(i, c):
            for ch in range(DD // L):
                v = out_v[i, pl.ds(ch * L, L)]
                out_v[i, pl.ds(ch * L, L)] = jnp.maximum(v, 0.0)
            return c

        lax.fori_loop(0, RN, fin, 0)

    pltpu.sync_copy(out_v, out_hbm.at[pl.ds(node_lo, RN)])


def _make_sc(relu):
    mesh = plsc.VectorSubcoreMesh(core_axis_name="c", subcore_axis_name="s",
                                  num_cores=NCORES, num_subcores=NSUB)
    return pl.kernel(
        functools.partial(_sc_body, relu),
        out_type=jax.ShapeDtypeStruct((NPAD, DD), jnp.float32),
        mesh=mesh,
        compiler_params=pltpu.CompilerParams(needs_layout_passes=False),
        scratch_types=[
            pltpu.VMEM((NPAD,), jnp.float32),
            pltpu.VMEM((RN,), jnp.float32),
            pltpu.VMEM((RN,), jnp.float32),
            pltpu.VMEM((RN,), jnp.float32),
            pltpu.VMEM((RN, DD), jnp.float32),
            pltpu.VMEM((CH_A,), jnp.int32),
            pltpu.VMEM((CH_A,), jnp.int32),
            pltpu.VMEM((CH_B,), jnp.int32),
            pltpu.VMEM((CH_B,), jnp.int32),
            pltpu.VMEM((CH_B,), jnp.int32),
            pltpu.VMEM((CH_B,), jnp.int32),
            pltpu.VMEM((CH_B, DD), jnp.float32),
            pltpu.VMEM((CH_B, DD), jnp.float32),
            pltpu.VMEM((48,), jnp.int32),
            pltpu.VMEM((DD,), jnp.float32),
            pltpu.SemaphoreType.DMA,
            pltpu.SemaphoreType.DMA,
        ],
    )


_sc_relu = _make_sc(True)
_sc_last = _make_sc(False)


def kernel(x, edge_index, W1s, W1d, a1s, a1d, b1, W2s, W2d, a2s, a2d, b2,
           W3s, W3d, a3s, a3d, b3):
    src = edge_index[0].astype(jnp.int32)
    dst = edge_index[1].astype(jnp.int32)
    order = jnp.argsort(dst)
    src_s = jnp.take(src, order)
    dst_s = jnp.take(dst, order)
    src_s = jnp.concatenate([src_s, jnp.zeros((EPAD - EE,), jnp.int32)])
    dst_s = jnp.concatenate([dst_s, jnp.zeros((EPAD - EE,), jnp.int32)])
    bounds = jnp.arange(0, NPAD + RN, RN, dtype=jnp.int32)
    starts = jnp.searchsorted(dst_s[:EE], bounds).astype(jnp.int32)
    starts = jnp.concatenate(
        [starts, jnp.full((48 - starts.shape[0],), EE, jnp.int32)])

    h = jnp.zeros((NPAD, DD), jnp.float32).at[:NN].set(x)
    layers = [
        (W1s, W1d, a1s, a1d, b1, _sc_relu),
        (W2s, W2d, a2s, a2d, b2, _sc_relu),
        (W3s, W3d, a3s, a3d, b3, _sc_last),
    ]
    for Ws, Wd, a_s, a_d, b, sc in layers:
        xs, asrc, adst = _tc_stage(h, Ws, Wd, a_s, a_d)
        h = sc(xs, asrc, adst, src_s, dst_s, starts, b)
    return h[:NN]


# speedup vs baseline: 24.7882x; 1.0392x over previous
import functools

import jax
import jax.numpy as jnp
from jax import lax
from jax.experimental import pallas as pl
from jax.experimental.pallas import tpu as pltpu
from jax.experimental.pallas import tpu_sc as plsc

NN = 10000
EE = 320000
DD = 128

L = 16
NCORES = 2
NSUB = 16
NW = NCORES * NSUB
RN = 320
NPAD = NW * RN
CH_A = 512
CH_B = 128
EPAD = EE + CH_A

_NEG = -1e30


def _iota16():
    return lax.iota(jnp.int32, L)


def _vgather(v, idx):
    dnums = lax.GatherDimensionNumbers(
        offset_dims=(), collapsed_slice_dims=(0,), start_index_map=(0,))
    return lax.gather(v, idx[:, None], dnums, (1,),
                      mode=lax.GatherScatterMode.PROMISE_IN_BOUNDS)


def _shift_up(v, k, fill):
    it = _iota16()
    src = jnp.maximum(it - k, 0)
    sh = _vgather(v, src)
    return jnp.where(it >= k, sh, fill)


_TC_BLK = 1024


def _tc_body(x_ref, ws_ref, wd_ref, avs_ref, avd_ref, xs_ref, as_ref, ad_ref):
    x = x_ref[...]
    xs = jnp.dot(x, ws_ref[...], preferred_element_type=jnp.float32)
    xd = jnp.dot(x, wd_ref[...], preferred_element_type=jnp.float32)
    xs_ref[...] = xs
    as_ref[...] = jnp.dot(xs, avs_ref[...], preferred_element_type=jnp.float32)
    ad_ref[...] = jnp.dot(xd, avd_ref[...], preferred_element_type=jnp.float32)


def _tc_stage(x, Ws, Wd, a_s, a_d):
    grid = (NPAD // _TC_BLK,)
    out = pl.pallas_call(
        _tc_body,
        grid=grid,
        in_specs=[
            pl.BlockSpec((_TC_BLK, DD), lambda i: (i, 0)),
            pl.BlockSpec((DD, DD), lambda i: (0, 0)),
            pl.BlockSpec((DD, DD), lambda i: (0, 0)),
            pl.BlockSpec((DD, 1), lambda i: (0, 0)),
            pl.BlockSpec((DD, 1), lambda i: (0, 0)),
        ],
        out_specs=[
            pl.BlockSpec((_TC_BLK, DD), lambda i: (i, 0)),
            pl.BlockSpec((_TC_BLK, 1), lambda i: (i, 0)),
            pl.BlockSpec((_TC_BLK, 1), lambda i: (i, 0)),
        ],
        out_shape=[
            jax.ShapeDtypeStruct((NPAD, DD), jnp.float32),
            jax.ShapeDtypeStruct((NPAD, 1), jnp.float32),
            jax.ShapeDtypeStruct((NPAD, 1), jnp.float32),
        ],
    )(x, Ws, Wd, a_s[:, None], a_d[:, None])
    xs, asrc, adst = out
    return xs, asrc[:, 0], adst[:, 0]


def _seg_merge(l, dst_v):
    pm = l
    ps = jnp.ones((L,), jnp.float32)
    it = _iota16()
    for k in (1, 2, 4, 8):
        m_sh = _shift_up(pm, k, _NEG)
        s_sh = _shift_up(ps, k, 0.0)
        d_sh = _shift_up(dst_v, k, -1)
        same = (d_sh == dst_v) & (it >= k)
        mn = jnp.maximum(pm, jnp.where(same, m_sh, _NEG))
        sn = ps * jnp.exp(pm - mn) + jnp.where(same, s_sh * jnp.exp(m_sh - mn), 0.0)
        pm, ps = mn, sn
    return pm, ps


def _run_end_mask(dst_v):
    it = _iota16()
    nxt = jnp.minimum(it + 1, L - 1)
    dn = _vgather(dst_v, nxt)
    return (dst_v != dn) | (it == L - 1)


def _sc_body(relu, xs_hbm, asrc_hbm, adst_hbm, src_hbm, dst_hbm, starts_hbm,
             b_hbm, out_hbm, asrc_v, adst_v, m_v, s_v, out_v, srcA, dstA,
             idxB0, idxB1, dstB0, dstB1, rows0, rows1, starts_v, bias_v,
             sem0, sem1):
    cid = lax.axis_index("c")
    sid = lax.axis_index("s")
    wid = sid * NCORES + cid
    node_lo = wid * RN

    pltpu.sync_copy(asrc_hbm, asrc_v)
    pltpu.sync_copy(adst_hbm.at[pl.ds(node_lo, RN)], adst_v)
    pltpu.sync_copy(starts_hbm, starts_v)
    pltpu.sync_copy(b_hbm, bias_v)

    e_lo = starts_v[pl.ds(wid, L)][0]
    e_hi = starts_v[pl.ds(wid + 1, L)][0]
    base = (e_lo // L) * L

    it = _iota16()
    negv = jnp.full((L,), _NEG, jnp.float32)
    zerov = jnp.zeros((L,), jnp.float32)

    def init_ms(i, c):
        m_v[pl.ds(i * L, L)] = negv
        s_v[pl.ds(i * L, L)] = zerov
        return c

    lax.fori_loop(0, RN // L, init_ms, 0)

    def init_out(i, c):
        for ch in range(DD // L):
            out_v[i, pl.ds(ch * L, L)] = bias_v[pl.ds(ch * L, L)]
        return c

    lax.fori_loop(0, RN, init_out, 0)

    n_a = (e_hi - base + CH_A - 1) // CH_A

    def passA(t, c):
        off = base + t * CH_A
        pltpu.sync_copy(src_hbm.at[pl.ds(off, CH_A)], srcA)
        pltpu.sync_copy(dst_hbm.at[pl.ds(off, CH_A)], dstA)

        def vloop(v, cc):
            src_v = srcA[pl.ds(v * L, L)]
            dst_v = dstA[pl.ds(v * L, L)]
            gidx = (off + v * L) + it
            valid = (gidx >= e_lo) & (gidx < e_hi)
            a1 = plsc.load_gather(asrc_v, [jnp.clip(src_v, 0, NPAD - 1)],
                                  mask=valid)
            dlc = jnp.clip(dst_v - node_lo, 0, RN - 1)
            a2 = plsc.load_gather(adst_v, [dlc], mask=valid)
            l = a1 + a2
            l = jnp.where(l > 0, l, 0.2 * l)
            l = jnp.where(valid, l, _NEG)
            pm, ps = _seg_merge(l, dst_v)
            wmask = _run_end_mask(dst_v) & valid
            old_m = plsc.load_gather(m_v, [dlc], mask=wmask)
            old_s = plsc.load_gather(s_v, [dlc], mask=wmask)
            mn = jnp.maximum(old_m, pm)
            sn = old_s * jnp.exp(old_m - mn) + ps * jnp.exp(pm - mn)
            plsc.store_scatter(m_v, [dlc], mn, mask=wmask)
            plsc.store_scatter(s_v, [dlc], sn, mask=wmask)
            return cc

        lax.fori_loop(0, CH_A // L, vloop, 0)
        return c

    lax.fori_loop(0, n_a, passA, 0)

    def inv_loop(i, c):
        sv = s_v[pl.ds(i * L, L)]
        s_v[pl.ds(i * L, L)] = 1.0 / (sv + 1e-16)
        return c

    lax.fori_loop(0, RN // L, inv_loop, 0)

    n_b = (e_hi - base + CH_B - 1) // CH_B
    idxBs = (idxB0, idxB1)
    dstBs = (dstB0, dstB1)
    rowss = (rows0, rows1)
    sems = (sem0, sem1)

    def _fetchB(t, b):
        off = base + t * CH_B
        pltpu.sync_copy(src_hbm.at[pl.ds(off, CH_B)], idxBs[b])
        pltpu.sync_copy(dst_hbm.at[pl.ds(off, CH_B)], dstBs[b])
        pltpu.make_async_copy(xs_hbm.at[idxBs[b]], rowss[b], sems[b]).start()

    @pl.when(n_b > 0)
    def _():
        _fetchB(0, 0)

    @pl.when(n_b > 1)
    def _():
        _fetchB(1, 1)

    def _consumeB(t, b):
        pltpu.make_async_copy(xs_hbm.at[idxBs[b]], rowss[b], sems[b]).wait()
        rows = rowss[b]

        def vloop(v, cc):
            src_v = idxBs[b][pl.ds(v * L, L)]
            dst_v = dstBs[b][pl.ds(v * L, L)]
            gidx = (base + t * CH_B + v * L) + it
            valid = (gidx >= e_lo) & (gidx < e_hi)
            a1 = plsc.load_gather(asrc_v, [jnp.clip(src_v, 0, NPAD - 1)],
                                  mask=valid)
            dlc = jnp.clip(dst_v - node_lo, 0, RN - 1)
            a2 = plsc.load_gather(adst_v, [dlc], mask=valid)
            l = a1 + a2
            l = jnp.where(l > 0, l, 0.2 * l)
            m_g = plsc.load_gather(m_v, [dlc], mask=valid)
            s_g = plsc.load_gather(s_v, [dlc], mask=valid)
            alpha = jnp.exp(l - m_g) * s_g
            alpha = jnp.where(valid, alpha, 0.0)
            rbase = v * L
            endi = _run_end_mask(dst_v).astype(jnp.int32)
            accs = [jnp.zeros((L,), jnp.float32) for _ in range(DD // L)]
            for j in range(L):
                av = lax.broadcast(alpha[j], (L,))
                accs = [acc + av * rows[rbase + j, pl.ds(ch * L, L)]
                        for ch, acc in enumerate(accs)]
                ej = endi[j] != 0
                d = dlc[j]

                @pl.when(ej)
                def _(accs=accs, d=d):
                    for ch in range(DD // L):
                        plsc.addupdate(out_v.at[d, pl.ds(ch * L, L)],
                                       accs[ch])

                eb = lax.broadcast(ej, (L,))
                accs = [jnp.where(eb, 0.0, acc) for acc in accs]
            return cc

        lax.fori_loop(0, CH_B // L, vloop, 0)

        @pl.when(t + 2 < n_b)
        def _():
            _fetchB(t + 2, b)

    def passB_pair(p, c):
        for b in range(2):
            t = p * 2 + b

            @pl.when(t < n_b)
            def _():
                _consumeB(t, b)

        return c

    lax.fori_loop(0, (n_b + 1) // 2, passB_pair, 0)

    if relu:
        def fin(i, c):
            for ch in range(DD // L):
                v = out_v[i, pl.ds(ch * L, L)]
                out_v[i, pl.ds(ch * L, L)] = jnp.maximum(v, 0.0)
            return c

        lax.fori_loop(0, RN, fin, 0)

    pltpu.sync_copy(out_v, out_hbm.at[pl.ds(node_lo, RN)])


def _make_sc(relu):
    mesh = plsc.VectorSubcoreMesh(core_axis_name="c", subcore_axis_name="s",
                                  num_cores=NCORES, num_subcores=NSUB)
    return pl.kernel(
        functools.partial(_sc_body, relu),
        out_type=jax.ShapeDtypeStruct((NPAD, DD), jnp.float32),
        mesh=mesh,
        compiler_params=pltpu.CompilerParams(needs_layout_passes=False),
        scratch_types=[
            pltpu.VMEM((NPAD,), jnp.float32),
            pltpu.VMEM((RN,), jnp.float32),
            pltpu.VMEM((RN,), jnp.float32),
            pltpu.VMEM((RN,), jnp.float32),
            pltpu.VMEM((RN, DD), jnp.float32),
            pltpu.VMEM((CH_A,), jnp.int32),
            pltpu.VMEM((CH_A,), jnp.int32),
            pltpu.VMEM((CH_B,), jnp.int32),
            pltpu.VMEM((CH_B,), jnp.int32),
            pltpu.VMEM((CH_B,), jnp.int32),
            pltpu.VMEM((CH_B,), jnp.int32),
            pltpu.VMEM((CH_B, DD), jnp.float32),
            pltpu.VMEM((CH_B, DD), jnp.float32),
            pltpu.VMEM((48,), jnp.int32),
            pltpu.VMEM((DD,), jnp.float32),
            pltpu.SemaphoreType.DMA,
            pltpu.SemaphoreType.DMA,
        ],
    )


_sc_relu = _make_sc(True)
_sc_last = _make_sc(False)


def kernel(x, edge_index, W1s, W1d, a1s, a1d, b1, W2s, W2d, a2s, a2d, b2,
           W3s, W3d, a3s, a3d, b3):
    src = edge_index[0].astype(jnp.int32)
    dst = edge_index[1].astype(jnp.int32)
    key = jnp.sort(dst * 16384 + src)
    src_s = key & 16383
    dst_s = key >> 14
    src_s = jnp.concatenate([src_s, jnp.zeros((EPAD - EE,), jnp.int32)])
    dst_s = jnp.concatenate([dst_s, jnp.zeros((EPAD - EE,), jnp.int32)])
    bounds = jnp.arange(0, NPAD + RN, RN, dtype=jnp.int32)
    starts = jnp.searchsorted(key, bounds * 16384).astype(jnp.int32)
    starts = jnp.concatenate(
        [starts, jnp.full((48 - starts.shape[0],), EE, jnp.int32)])

    h = jnp.zeros((NPAD, DD), jnp.float32).at[:NN].set(x)
    layers = [
        (W1s, W1d, a1s, a1d, b1, _sc_relu),
        (W2s, W2d, a2s, a2d, b2, _sc_relu),
        (W3s, W3d, a3s, a3d, b3, _sc_last),
    ]
    for Ws, Wd, a_s, a_d, b, sc in layers:
        xs, asrc, adst = _tc_stage(h, Ws, Wd, a_s, a_d)
        h = sc(xs, asrc, adst, src_s, dst_s, starts, b)
    return h[:NN]


# speedup vs baseline: 25.4068x; 1.0250x over previous
import functools

import jax
import jax.numpy as jnp
from jax import lax
from jax.experimental import pallas as pl
from jax.experimental.pallas import tpu as pltpu
from jax.experimental.pallas import tpu_sc as plsc

NN = 10000
EE = 320000
DD = 128

L = 16
NCORES = 2
NSUB = 16
NW = NCORES * NSUB
RN = 320
NPAD = NW * RN
CH_A = 2048
CH_B = 128
EPAD = EE + CH_A

_NEG = -1e30


def _iota16():
    return lax.iota(jnp.int32, L)


def _vgather(v, idx):
    dnums = lax.GatherDimensionNumbers(
        offset_dims=(), collapsed_slice_dims=(0,), start_index_map=(0,))
    return lax.gather(v, idx[:, None], dnums, (1,),
                      mode=lax.GatherScatterMode.PROMISE_IN_BOUNDS)


def _shift_up(v, k, fill):
    it = _iota16()
    src = jnp.maximum(it - k, 0)
    sh = _vgather(v, src)
    return jnp.where(it >= k, sh, fill)


_TC_BLK = 1024


def _tc_body(x_ref, ws_ref, wd_ref, avs_ref, avd_ref, xs_ref, as_ref, ad_ref):
    x = x_ref[...]
    xs = jnp.dot(x, ws_ref[...], preferred_element_type=jnp.float32)
    xd = jnp.dot(x, wd_ref[...], preferred_element_type=jnp.float32)
    xs_ref[...] = xs
    as_ref[...] = jnp.dot(xs, avs_ref[...], preferred_element_type=jnp.float32)
    ad_ref[...] = jnp.dot(xd, avd_ref[...], preferred_element_type=jnp.float32)


def _tc_stage(x, Ws, Wd, a_s, a_d):
    grid = (NPAD // _TC_BLK,)
    out = pl.pallas_call(
        _tc_body,
        grid=grid,
        in_specs=[
            pl.BlockSpec((_TC_BLK, DD), lambda i: (i, 0)),
            pl.BlockSpec((DD, DD), lambda i: (0, 0)),
            pl.BlockSpec((DD, DD), lambda i: (0, 0)),
            pl.BlockSpec((DD, 1), lambda i: (0, 0)),
            pl.BlockSpec((DD, 1), lambda i: (0, 0)),
        ],
        out_specs=[
            pl.BlockSpec((_TC_BLK, DD), lambda i: (i, 0)),
            pl.BlockSpec((_TC_BLK, 1), lambda i: (i, 0)),
            pl.BlockSpec((_TC_BLK, 1), lambda i: (i, 0)),
        ],
        out_shape=[
            jax.ShapeDtypeStruct((NPAD, DD), jnp.float32),
            jax.ShapeDtypeStruct((NPAD, 1), jnp.float32),
            jax.ShapeDtypeStruct((NPAD, 1), jnp.float32),
        ],
    )(x, Ws, Wd, a_s[:, None], a_d[:, None])
    xs, asrc, adst = out
    return xs, asrc[:, 0], adst[:, 0]


def _seg_merge(l, dst_v):
    pm = l
    ps = jnp.ones((L,), jnp.float32)
    it = _iota16()
    for k in (1, 2, 4, 8):
        m_sh = _shift_up(pm, k, _NEG)
        s_sh = _shift_up(ps, k, 0.0)
        d_sh = _shift_up(dst_v, k, -1)
        same = (d_sh == dst_v) & (it >= k)
        mn = jnp.maximum(pm, jnp.where(same, m_sh, _NEG))
        sn = ps * jnp.exp(pm - mn) + jnp.where(same, s_sh * jnp.exp(m_sh - mn), 0.0)
        pm, ps = mn, sn
    return pm, ps


def _run_end_mask(dst_v):
    it = _iota16()
    nxt = jnp.minimum(it + 1, L - 1)
    dn = _vgather(dst_v, nxt)
    return (dst_v != dn) | (it == L - 1)


def _sc_body(relu, xs_hbm, asrc_hbm, adst_hbm, src_hbm, dst_hbm, starts_hbm,
             b_hbm, out_hbm, asrc_v, adst_v, m_v, s_v, out_v, srcA, dstA,
             idxB0, idxB1, dstB0, dstB1, rows0, rows1, starts_v, bias_v,
             sem0, sem1):
    cid = lax.axis_index("c")
    sid = lax.axis_index("s")
    wid = sid * NCORES + cid
    node_lo = wid * RN

    pltpu.sync_copy(asrc_hbm, asrc_v)
    pltpu.sync_copy(adst_hbm.at[pl.ds(node_lo, RN)], adst_v)
    pltpu.sync_copy(starts_hbm, starts_v)
    pltpu.sync_copy(b_hbm, bias_v)

    e_lo = starts_v[pl.ds(wid, L)][0]
    e_hi = starts_v[pl.ds(wid + 1, L)][0]
    base = (e_lo // L) * L

    it = _iota16()
    negv = jnp.full((L,), _NEG, jnp.float32)
    zerov = jnp.zeros((L,), jnp.float32)

    def init_ms(i, c):
        m_v[pl.ds(i * L, L)] = negv
        s_v[pl.ds(i * L, L)] = zerov
        return c

    lax.fori_loop(0, RN // L, init_ms, 0)

    def init_out(i, c):
        for ch in range(DD // L):
            out_v[i, pl.ds(ch * L, L)] = bias_v[pl.ds(ch * L, L)]
        return c

    lax.fori_loop(0, RN, init_out, 0)

    n_a = (e_hi - base + CH_A - 1) // CH_A

    def passA(t, c):
        off = base + t * CH_A
        pltpu.sync_copy(src_hbm.at[pl.ds(off, CH_A)], srcA)
        pltpu.sync_copy(dst_hbm.at[pl.ds(off, CH_A)], dstA)

        def vloop(v, cc):
            src_v = srcA[pl.ds(v * L, L)]
            dst_v = dstA[pl.ds(v * L, L)]
            gidx = (off + v * L) + it
            valid = (gidx >= e_lo) & (gidx < e_hi)
            a1 = plsc.load_gather(asrc_v, [jnp.clip(src_v, 0, NPAD - 1)],
                                  mask=valid)
            dlc = jnp.clip(dst_v - node_lo, 0, RN - 1)
            a2 = plsc.load_gather(adst_v, [dlc], mask=valid)
            l = a1 + a2
            l = jnp.where(l > 0, l, 0.2 * l)
            l = jnp.where(valid, l, _NEG)
            pm, ps = _seg_merge(l, dst_v)
            wmask = _run_end_mask(dst_v) & valid
            old_m = plsc.load_gather(m_v, [dlc], mask=wmask)
            old_s = plsc.load_gather(s_v, [dlc], mask=wmask)
            mn = jnp.maximum(old_m, pm)
            sn = old_s * jnp.exp(old_m - mn) + ps * jnp.exp(pm - mn)
            plsc.store_scatter(m_v, [dlc], mn, mask=wmask)
            plsc.store_scatter(s_v, [dlc], sn, mask=wmask)
            return cc

        lax.fori_loop(0, CH_A // L, vloop, 0)
        return c

    lax.fori_loop(0, n_a, passA, 0)

    def inv_loop(i, c):
        sv = s_v[pl.ds(i * L, L)]
        s_v[pl.ds(i * L, L)] = 1.0 / (sv + 1e-16)
        return c

    lax.fori_loop(0, RN // L, inv_loop, 0)

    n_b = (e_hi - base + CH_B - 1) // CH_B
    idxBs = (idxB0, idxB1)
    dstBs = (dstB0, dstB1)
    rowss = (rows0, rows1)
    sems = (sem0, sem1)

    def _fetchB(t, b):
        off = base + t * CH_B
        pltpu.sync_copy(src_hbm.at[pl.ds(off, CH_B)], idxBs[b])
        pltpu.sync_copy(dst_hbm.at[pl.ds(off, CH_B)], dstBs[b])
        pltpu.make_async_copy(xs_hbm.at[idxBs[b]], rowss[b], sems[b]).start()

    @pl.when(n_b > 0)
    def _():
        _fetchB(0, 0)

    @pl.when(n_b > 1)
    def _():
        _fetchB(1, 1)

    def _consumeB(t, b):
        pltpu.make_async_copy(xs_hbm.at[idxBs[b]], rowss[b], sems[b]).wait()
        rows = rowss[b]

        def vloop(v, cc):
            src_v = idxBs[b][pl.ds(v * L, L)]
            dst_v = dstBs[b][pl.ds(v * L, L)]
            gidx = (base + t * CH_B + v * L) + it
            valid = (gidx >= e_lo) & (gidx < e_hi)
            a1 = plsc.load_gather(asrc_v, [jnp.clip(src_v, 0, NPAD - 1)],
                                  mask=valid)
            dlc = jnp.clip(dst_v - node_lo, 0, RN - 1)
            a2 = plsc.load_gather(adst_v, [dlc], mask=valid)
            l = a1 + a2
            l = jnp.where(l > 0, l, 0.2 * l)
            m_g = plsc.load_gather(m_v, [dlc], mask=valid)
            s_g = plsc.load_gather(s_v, [dlc], mask=valid)
            alpha = jnp.exp(l - m_g) * s_g
            alpha = jnp.where(valid, alpha, 0.0)
            rbase = v * L
            endi = _run_end_mask(dst_v).astype(jnp.int32)
            accs = [jnp.zeros((L,), jnp.float32) for _ in range(DD // L)]
            for j in range(L):
                av = lax.broadcast(alpha[j], (L,))
                accs = [acc + av * rows[rbase + j, pl.ds(ch * L, L)]
                        for ch, acc in enumerate(accs)]
                ej = endi[j] != 0
                d = dlc[j]

                @pl.when(ej)
                def _(accs=accs, d=d):
                    for ch in range(DD // L):
                        plsc.addupdate(out_v.at[d, pl.ds(ch * L, L)],
                                       accs[ch])

                eb = lax.broadcast(ej, (L,))
                accs = [jnp.where(eb, 0.0, acc) for acc in accs]
            return cc

        lax.fori_loop(0, CH_B // L, vloop, 0)

        @pl.when(t + 2 < n_b)
        def _():
            _fetchB(t + 2, b)

    def passB_pair(p, c):
        for b in range(2):
            t = p * 2 + b

            @pl.when(t < n_b)
            def _():
                _consumeB(t, b)

        return c

    lax.fori_loop(0, (n_b + 1) // 2, passB_pair, 0)

    if relu:
        def fin(i, c):
            for ch in range(DD // L):
                v = out_v[i, pl.ds(ch * L, L)]
                out_v[i, pl.ds(ch * L, L)] = jnp.maximum(v, 0.0)
            return c

        lax.fori_loop(0, RN, fin, 0)

    pltpu.sync_copy(out_v, out_hbm.at[pl.ds(node_lo, RN)])


def _make_sc(relu):
    mesh = plsc.VectorSubcoreMesh(core_axis_name="c", subcore_axis_name="s",
                                  num_cores=NCORES, num_subcores=NSUB)
    return pl.kernel(
        functools.partial(_sc_body, relu),
        out_type=jax.ShapeDtypeStruct((NPAD, DD), jnp.float32),
        mesh=mesh,
        compiler_params=pltpu.CompilerParams(needs_layout_passes=False),
        scratch_types=[
            pltpu.VMEM((NPAD,), jnp.float32),
            pltpu.VMEM((RN,), jnp.float32),
            pltpu.VMEM((RN,), jnp.float32),
            pltpu.VMEM((RN,), jnp.float32),
            pltpu.VMEM((RN, DD), jnp.float32),
            pltpu.VMEM((CH_A,), jnp.int32),
            pltpu.VMEM((CH_A,), jnp.int32),
            pltpu.VMEM((CH_B,), jnp.int32),
            pltpu.VMEM((CH_B,), jnp.int32),
            pltpu.VMEM((CH_B,), jnp.int32),
            pltpu.VMEM((CH_B,), jnp.int32),
            pltpu.VMEM((CH_B, DD), jnp.float32),
            pltpu.VMEM((CH_B, DD), jnp.float32),
            pltpu.VMEM((48,), jnp.int32),
            pltpu.VMEM((DD,), jnp.float32),
            pltpu.SemaphoreType.DMA,
            pltpu.SemaphoreType.DMA,
        ],
    )


_sc_relu = _make_sc(True)
_sc_last = _make_sc(False)


def kernel(x, edge_index, W1s, W1d, a1s, a1d, b1, W2s, W2d, a2s, a2d, b2,
           W3s, W3d, a3s, a3d, b3):
    src = edge_index[0].astype(jnp.int32)
    dst = edge_index[1].astype(jnp.int32)
    key = jnp.sort(dst * 16384 + src)
    src_s = key & 16383
    dst_s = key >> 14
    src_s = jnp.concatenate([src_s, jnp.zeros((EPAD - EE,), jnp.int32)])
    dst_s = jnp.concatenate([dst_s, jnp.zeros((EPAD - EE,), jnp.int32)])
    bounds = jnp.arange(0, NPAD + RN, RN, dtype=jnp.int32)
    starts = jnp.searchsorted(key, bounds * 16384).astype(jnp.int32)
    starts = jnp.concatenate(
        [starts, jnp.full((48 - starts.shape[0],), EE, jnp.int32)])

    h = jnp.zeros((NPAD, DD), jnp.float32).at[:NN].set(x)
    layers = [
        (W1s, W1d, a1s, a1d, b1, _sc_relu),
        (W2s, W2d, a2s, a2d, b2, _sc_relu),
        (W3s, W3d, a3s, a3d, b3, _sc_last),
    ]
    for Ws, Wd, a_s, a_d, b, sc in layers:
        xs, asrc, adst = _tc_stage(h, Ws, Wd, a_s, a_d)
        h = sc(xs, asrc, adst, src_s, dst_s, starts, b)
    return h[:NN]


# speedup vs baseline: 25.5874x; 1.0071x over previous
import functools

import jax
import jax.numpy as jnp
from jax import lax
from jax.experimental import pallas as pl
from jax.experimental.pallas import tpu as pltpu
from jax.experimental.pallas import tpu_sc as plsc

NN = 10000
EE = 320000
DD = 128

L = 16
NCORES = 2
NSUB = 16
NW = NCORES * NSUB
RN = 320
NPAD = NW * RN
CH_A = 4096
CH_B = 128
EPAD = EE + CH_A

_NEG = -1e30


def _iota16():
    return lax.iota(jnp.int32, L)


def _vgather(v, idx):
    dnums = lax.GatherDimensionNumbers(
        offset_dims=(), collapsed_slice_dims=(0,), start_index_map=(0,))
    return lax.gather(v, idx[:, None], dnums, (1,),
                      mode=lax.GatherScatterMode.PROMISE_IN_BOUNDS)


def _shift_up(v, k, fill):
    it = _iota16()
    src = jnp.maximum(it - k, 0)
    sh = _vgather(v, src)
    return jnp.where(it >= k, sh, fill)


_TC_BLK = 1024


def _tc_body(x_ref, ws_ref, wd_ref, avs_ref, avd_ref, xs_ref, as_ref, ad_ref):
    x = x_ref[...]
    xs = jnp.dot(x, ws_ref[...], preferred_element_type=jnp.float32)
    xd = jnp.dot(x, wd_ref[...], preferred_element_type=jnp.float32)
    xs_ref[...] = xs
    as_ref[...] = jnp.dot(xs, avs_ref[...], preferred_element_type=jnp.float32)
    ad_ref[...] = jnp.dot(xd, avd_ref[...], preferred_element_type=jnp.float32)


def _tc_stage(x, Ws, Wd, a_s, a_d):
    grid = (NPAD // _TC_BLK,)
    out = pl.pallas_call(
        _tc_body,
        grid=grid,
        in_specs=[
            pl.BlockSpec((_TC_BLK, DD), lambda i: (i, 0)),
            pl.BlockSpec((DD, DD), lambda i: (0, 0)),
            pl.BlockSpec((DD, DD), lambda i: (0, 0)),
            pl.BlockSpec((DD, 1), lambda i: (0, 0)),
            pl.BlockSpec((DD, 1), lambda i: (0, 0)),
        ],
        out_specs=[
            pl.BlockSpec((_TC_BLK, DD), lambda i: (i, 0)),
            pl.BlockSpec((_TC_BLK, 1), lambda i: (i, 0)),
            pl.BlockSpec((_TC_BLK, 1), lambda i: (i, 0)),
        ],
        out_shape=[
            jax.ShapeDtypeStruct((NPAD, DD), jnp.float32),
            jax.ShapeDtypeStruct((NPAD, 1), jnp.float32),
            jax.ShapeDtypeStruct((NPAD, 1), jnp.float32),
        ],
    )(x, Ws, Wd, a_s[:, None], a_d[:, None])
    xs, asrc, adst = out
    return xs, asrc[:, 0], adst[:, 0]


def _seg_merge(l, dst_v):
    pm = l
    ps = jnp.ones((L,), jnp.float32)
    it = _iota16()
    for k in (1, 2, 4, 8):
        m_sh = _shift_up(pm, k, _NEG)
        s_sh = _shift_up(ps, k, 0.0)
        d_sh = _shift_up(dst_v, k, -1)
        same = (d_sh == dst_v) & (it >= k)
        mn = jnp.maximum(pm, jnp.where(same, m_sh, _NEG))
        sn = ps * jnp.exp(pm - mn) + jnp.where(same, s_sh * jnp.exp(m_sh - mn), 0.0)
        pm, ps = mn, sn
    return pm, ps


def _run_end_mask(dst_v):
    it = _iota16()
    nxt = jnp.minimum(it + 1, L - 1)
    dn = _vgather(dst_v, nxt)
    return (dst_v != dn) | (it == L - 1)


def _sc_body(relu, xs_hbm, asrc_hbm, adst_hbm, src_hbm, dst_hbm, starts_hbm,
             b_hbm, out_hbm, asrc_v, adst_v, m_v, s_v, out_v, srcA, dstA,
             idxB0, idxB1, dstB0, dstB1, rows0, rows1, starts_v, bias_v,
             sem0, sem1):
    cid = lax.axis_index("c")
    sid = lax.axis_index("s")
    wid = sid * NCORES + cid
    node_lo = wid * RN

    pltpu.sync_copy(asrc_hbm, asrc_v)
    pltpu.sync_copy(adst_hbm.at[pl.ds(node_lo, RN)], adst_v)
    pltpu.sync_copy(starts_hbm, starts_v)
    pltpu.sync_copy(b_hbm, bias_v)

    e_lo = starts_v[pl.ds(wid, L)][0]
    e_hi = starts_v[pl.ds(wid + 1, L)][0]
    base = (e_lo // L) * L

    it = _iota16()
    negv = jnp.full((L,), _NEG, jnp.float32)
    zerov = jnp.zeros((L,), jnp.float32)

    def init_ms(i, c):
        m_v[pl.ds(i * L, L)] = negv
        s_v[pl.ds(i * L, L)] = zerov
        return c

    lax.fori_loop(0, RN // L, init_ms, 0)

    def init_out(i, c):
        for ch in range(DD // L):
            out_v[i, pl.ds(ch * L, L)] = bias_v[pl.ds(ch * L, L)]
        return c

    lax.fori_loop(0, RN, init_out, 0)

    n_a = (e_hi - base + CH_A - 1) // CH_A

    def passA(t, c):
        off = base + t * CH_A
        pltpu.sync_copy(src_hbm.at[pl.ds(off, CH_A)], srcA)
        pltpu.sync_copy(dst_hbm.at[pl.ds(off, CH_A)], dstA)

        def vloop(v, cc):
            src_v = srcA[pl.ds(v * L, L)]
            dst_v = dstA[pl.ds(v * L, L)]
            gidx = (off + v * L) + it
            valid = (gidx >= e_lo) & (gidx < e_hi)
            a1 = plsc.load_gather(asrc_v, [jnp.clip(src_v, 0, NPAD - 1)],
                                  mask=valid)
            dlc = jnp.clip(dst_v - node_lo, 0, RN - 1)
            a2 = plsc.load_gather(adst_v, [dlc], mask=valid)
            l = a1 + a2
            l = jnp.where(l > 0, l, 0.2 * l)
            l = jnp.where(valid, l, _NEG)
            pm, ps = _seg_merge(l, dst_v)
            wmask = _run_end_mask(dst_v) & valid
            old_m = plsc.load_gather(m_v, [dlc], mask=wmask)
            old_s = plsc.load_gather(s_v, [dlc], mask=wmask)
            mn = jnp.maximum(old_m, pm)
            sn = old_s * jnp.exp(old_m - mn) + ps * jnp.exp(pm - mn)
            plsc.store_scatter(m_v, [dlc], mn, mask=wmask)
            plsc.store_scatter(s_v, [dlc], sn, mask=wmask)
            return cc

        lax.fori_loop(0, CH_A // L, vloop, 0)
        return c

    lax.fori_loop(0, n_a, passA, 0)

    def inv_loop(i, c):
        sv = s_v[pl.ds(i * L, L)]
        s_v[pl.ds(i * L, L)] = 1.0 / (sv + 1e-16)
        return c

    lax.fori_loop(0, RN // L, inv_loop, 0)

    n_b = (e_hi - base + CH_B - 1) // CH_B
    idxBs = (idxB0, idxB1)
    dstBs = (dstB0, dstB1)
    rowss = (rows0, rows1)
    sems = (sem0, sem1)

    def _fetchB(t, b):
        off = base + t * CH_B
        pltpu.sync_copy(src_hbm.at[pl.ds(off, CH_B)], idxBs[b])
        pltpu.sync_copy(dst_hbm.at[pl.ds(off, CH_B)], dstBs[b])
        pltpu.make_async_copy(xs_hbm.at[idxBs[b]], rowss[b], sems[b]).start()

    @pl.when(n_b > 0)
    def _():
        _fetchB(0, 0)

    @pl.when(n_b > 1)
    def _():
        _fetchB(1, 1)

    def _consumeB(t, b):
        pltpu.make_async_copy(xs_hbm.at[idxBs[b]], rowss[b], sems[b]).wait()
        rows = rowss[b]

        def vloop(v, cc):
            src_v = idxBs[b][pl.ds(v * L, L)]
            dst_v = dstBs[b][pl.ds(v * L, L)]
            gidx = (base + t * CH_B + v * L) + it
            valid = (gidx >= e_lo) & (gidx < e_hi)
            a1 = plsc.load_gather(asrc_v, [jnp.clip(src_v, 0, NPAD - 1)],
                                  mask=valid)
            dlc = jnp.clip(dst_v - node_lo, 0, RN - 1)
            a2 = plsc.load_gather(adst_v, [dlc], mask=valid)
            l = a1 + a2
            l = jnp.where(l > 0, l, 0.2 * l)
            m_g = plsc.load_gather(m_v, [dlc], mask=valid)
            s_g = plsc.load_gather(s_v, [dlc], mask=valid)
            alpha = jnp.exp(l - m_g) * s_g
            alpha = jnp.where(valid, alpha, 0.0)
            rbase = v * L
            endi = _run_end_mask(dst_v).astype(jnp.int32)
            accs = [jnp.zeros((L,), jnp.float32) for _ in range(DD // L)]
            for j in range(L):
                av = lax.broadcast(alpha[j], (L,))
                accs = [acc + av * rows[rbase + j, pl.ds(ch * L, L)]
                        for ch, acc in enumerate(accs)]
                ej = endi[j] != 0
                d = dlc[j]

                @pl.when(ej)
                def _(accs=accs, d=d):
                    for ch in range(DD // L):
                        plsc.addupdate(out_v.at[d, pl.ds(ch * L, L)],
                                       accs[ch])

                eb = lax.broadcast(ej, (L,))
                accs = [jnp.where(eb, 0.0, acc) for acc in accs]
            return cc

        lax.fori_loop(0, CH_B // L, vloop, 0)

        @pl.when(t + 2 < n_b)
        def _():
            _fetchB(t + 2, b)

    def passB_pair(p, c):
        for b in range(2):
            t = p * 2 + b

            @pl.when(t < n_b)
            def _():
                _consumeB(t, b)

        return c

    lax.fori_loop(0, (n_b + 1) // 2, passB_pair, 0)

    if relu:
        def fin(i, c):
            for ch in range(DD // L):
                v = out_v[i, pl.ds(ch * L, L)]
                out_v[i, pl.ds(ch * L, L)] = jnp.maximum(v, 0.0)
            return c

        lax.fori_loop(0, RN, fin, 0)

    pltpu.sync_copy(out_v, out_hbm.at[pl.ds(node_lo, RN)])


def _make_sc(relu):
    mesh = plsc.VectorSubcoreMesh(core_axis_name="c", subcore_axis_name="s",
                                  num_cores=NCORES, num_subcores=NSUB)
    return pl.kernel(
        functools.partial(_sc_body, relu),
        out_type=jax.ShapeDtypeStruct((NPAD, DD), jnp.float32),
        mesh=mesh,
        compiler_params=pltpu.CompilerParams(needs_layout_passes=False),
        scratch_types=[
            pltpu.VMEM((NPAD,), jnp.float32),
            pltpu.VMEM((RN,), jnp.float32),
            pltpu.VMEM((RN,), jnp.float32),
            pltpu.VMEM((RN,), jnp.float32),
            pltpu.VMEM((RN, DD), jnp.float32),
            pltpu.VMEM((CH_A,), jnp.int32),
            pltpu.VMEM((CH_A,), jnp.int32),
            pltpu.VMEM((CH_B,), jnp.int32),
            pltpu.VMEM((CH_B,), jnp.int32),
            pltpu.VMEM((CH_B,), jnp.int32),
            pltpu.VMEM((CH_B,), jnp.int32),
            pltpu.VMEM((CH_B, DD), jnp.float32),
            pltpu.VMEM((CH_B, DD), jnp.float32),
            pltpu.VMEM((48,), jnp.int32),
            pltpu.VMEM((DD,), jnp.float32),
            pltpu.SemaphoreType.DMA,
            pltpu.SemaphoreType.DMA,
        ],
    )


_sc_relu = _make_sc(True)
_sc_last = _make_sc(False)


def kernel(x, edge_index, W1s, W1d, a1s, a1d, b1, W2s, W2d, a2s, a2d, b2,
           W3s, W3d, a3s, a3d, b3):
    src = edge_index[0].astype(jnp.int32)
    dst = edge_index[1].astype(jnp.int32)
    key = jnp.sort(dst * 16384 + src)
    src_s = key & 16383
    dst_s = key >> 14
    src_s = jnp.concatenate([src_s, jnp.zeros((EPAD - EE,), jnp.int32)])
    dst_s = jnp.concatenate([dst_s, jnp.zeros((EPAD - EE,), jnp.int32)])
    bounds = jnp.arange(0, NPAD + RN, RN, dtype=jnp.int32)
    starts = jnp.searchsorted(key, bounds * 16384).astype(jnp.int32)
    starts = jnp.concatenate(
        [starts, jnp.full((48 - starts.shape[0],), EE, jnp.int32)])

    h = jnp.zeros((NPAD, DD), jnp.float32).at[:NN].set(x)
    layers = [
        (W1s, W1d, a1s, a1d, b1, _sc_relu),
        (W2s, W2d, a2s, a2d, b2, _sc_relu),
        (W3s, W3d, a3s, a3d, b3, _sc_last),
    ]
    for Ws, Wd, a_s, a_d, b, sc in layers:
        xs, asrc, adst = _tc_stage(h, Ws, Wd, a_s, a_d)
        h = sc(xs, asrc, adst, src_s, dst_s, starts, b)
    return h[:NN]
